# baseline (speedup 1.0000x reference)
import jax
import jax.numpy as jnp
from jax import lax
from jax.experimental import pallas as pl
from jax.experimental.pallas import tpu as pltpu
from jax.experimental.pallas import tpu_sc as plsc

GS = 128
F = 32
N = 2000000
V = GS * GS * GS

NC = 2
NS = 16
NW = NC * NS

C = 640
NIDX = C // 128
NCHUNKS = N // C
NMAXH = (NCHUNKS + 2 * NW - 1) // (2 * NW)

YB = 32
IRB = 16
NPROWS = N // 128
NPROWS_PAD = ((NPROWS + IRB - 1) // IRB) * IRB

NPB = N // 128
_RND = 8388608.0


def _tr_body(g_ref, t_ref):
    for yq in range(YB // 4):
        parts = [
            jnp.transpose(g_ref[0, yq * 4 + p], (1, 0)) for p in range(4)
        ]
        t_ref[yq] = jnp.concatenate(parts, axis=1)


def _idx_body(p_ref, o_ref):
    def rnd(t):
        t = jnp.clip(t * (GS - 1.0), 0.0, GS - 1.0)
        return (t + _RND) - _RND

    for r in range(IRB):
        sl = pl.ds(r * 128, 128)
        x = rnd(p_ref[0:1, sl])
        y = rnd(p_ref[1:2, sl])
        z = rnd(p_ref[2:3, sl])
        yq = jnp.floor(y * 0.25)
        yr = y - yq * 4.0
        o_ref[r : r + 1, :] = (
            x * 16384.0 + yq * 512.0 + z * 4.0 + yr
        ).astype(jnp.int32)


def _sc_body(idx_hbm, table_hbm, out_hbm, idx_v, rows_v, trows_v,
             si0, si1, sg0, sg1, so0, so1):
    wid = lax.axis_index("s") * NC + lax.axis_index("c")
    sem_in = (si0, si1)
    sem_g = (sg0, sg1)
    sem_out = (so0, so1)
    lane = lax.iota(jnp.int32, 16)

    def in_copy(k, s):
        return pltpu.make_async_copy(
            idx_hbm.at[pl.ds((wid + k * NW) * NIDX, NIDX)],
            idx_v.at[s],
            sem_in[s],
        )

    def gather_copies(s):
        return [
            pltpu.make_async_copy(
                table_hbm.at[idx_v.at[s, j]],
                rows_v.at[s, pl.ds(j * 128, 128)],
                sem_g[s],
            )
            for j in range(NIDX)
        ]

    def out_copies(k, s):
        c = wid + k * NW
        return [
            pltpu.make_async_copy(
                trows_v.at[s, fb],
                out_hbm.at[fb, pl.ds(c * NIDX, NIDX)],
                sem_out[s],
            )
            for fb in range(4)
        ]

    def drain_out(s):
        for fb in range(4):
            pltpu.make_async_copy(
                out_hbm.at[fb, pl.ds(0, NIDX)], trows_v.at[s, fb], sem_out[s]
            ).wait()

    def transpose_chunk(s):
        def fbody(f, carry):
            fb = f // 8
            fr = f % 8
            fbv = jnp.broadcast_to(fb, (16,))
            fv = jnp.broadcast_to(f, (16,))
            for pbi in range(NIDX):
                for g in range(8):
                    pidx = pbi * 128 + g * 16 + lane
                    val = plsc.load_gather(
                        rows_v, [jnp.broadcast_to(s, (16,)), pidx, fv]
                    )
                    plsc.store_scatter(
                        trows_v,
                        [
                            jnp.broadcast_to(s, (16,)),
                            fbv,
                            jnp.broadcast_to(pbi, (16,)),
                            fr * 128 + g * 16 + lane,
                        ],
                        val,
                    )
            return carry

        lax.fori_loop(0, F, fbody, 0)

    def valid(k):
        return wid + k * NW < NCHUNKS

    in_copy(0, 0).start()
    in_copy(1, 1).start()

    def outer(io, carry):
        for b in range(2):
            k = io * 2 + b

            @pl.when(valid(k))
            def _():
                in_copy(k, b).wait()

                @pl.when(io > 0)
                def _():
                    drain_out(b)

                for cp in gather_copies(b):
                    cp.start()
                for cp in gather_copies(b):
                    cp.wait()

                @pl.when(valid(k + 2))
                def _():
                    in_copy(k + 2, b).start()

                transpose_chunk(b)
                for cp in out_copies(k, b):
                    cp.start()

        return carry

    lax.fori_loop(0, NMAXH, outer, 0)

    for b in range(2):
        drain_out(b)


def _run(points, grid):
    g2 = jnp.transpose(grid, (0, 1, 3, 2))
    pts_t = jnp.transpose(points, (1, 0))

    table = pl.pallas_call(
        _tr_body,
        grid=(GS, GS // YB),
        in_specs=[
            pl.BlockSpec((1, YB, F, GS), lambda i, j: (i, j, 0, 0)),
        ],
        out_specs=pl.BlockSpec(
            (YB // 4, GS, 4 * F), lambda i, j: (i * (GS // YB) + j, 0, 0)
        ),
        out_shape=jax.ShapeDtypeStruct((GS * GS // 4, GS, 4 * F), jnp.float32),
    )(g2)
    table = table.reshape(V, F)

    idx = pl.pallas_call(
        _idx_body,
        grid=(NPROWS_PAD // IRB,),
        in_specs=[pl.BlockSpec((3, IRB * 128), lambda i: (0, i))],
        out_specs=pl.BlockSpec((IRB, 128), lambda i: (i, 0)),
        out_shape=jax.ShapeDtypeStruct((NPROWS_PAD, 128), jnp.int32),
    )(pts_t)

    mesh = plsc.VectorSubcoreMesh(core_axis_name="c", subcore_axis_name="s")
    run = pl.kernel(
        _sc_body,
        out_type=jax.ShapeDtypeStruct((4, NPROWS, 8 * 128), jnp.float32),
        mesh=mesh,
        compiler_params=pltpu.CompilerParams(
            needs_layout_passes=False, use_tc_tiling_on_sc=False
        ),
        scratch_types=[
            pltpu.VMEM((2, NIDX, 128), jnp.int32),
            pltpu.VMEM((2, C, F), jnp.float32),
            pltpu.VMEM((2, 4, NIDX, 8 * 128), jnp.float32),
            pltpu.SemaphoreType.DMA,
            pltpu.SemaphoreType.DMA,
            pltpu.SemaphoreType.DMA,
            pltpu.SemaphoreType.DMA,
            pltpu.SemaphoreType.DMA,
            pltpu.SemaphoreType.DMA,
        ],
    )
    out4 = run(idx, table)
    out4 = out4.reshape(4, NPROWS, 8, 128)
    outt = jnp.transpose(out4, (0, 2, 1, 3)).reshape(F, N)
    return jnp.transpose(outt, (1, 0))


_run_jit = jax.jit(_run)


def kernel(points, grid):
    return _run_jit(points, grid)


# speedup vs baseline: 1.8911x; 1.8911x over previous
import jax
import jax.numpy as jnp
from jax import lax
from jax.experimental import pallas as pl
from jax.experimental.pallas import tpu as pltpu
from jax.experimental.pallas import tpu_sc as plsc

GS = 128
F = 32
N = 2000000
V = GS * GS * GS

NC = 2
NS = 16
NW = NC * NS

C = 640
NIDX = C // 128
NCHUNKS = N // C
NMAXH = (NCHUNKS + 2 * NW - 1) // (2 * NW)

YB = 32
IRB = 16
NPROWS = N // 128
NPROWS_PAD = ((NPROWS + IRB - 1) // IRB) * IRB

NPB = N // 128
_RND = 8388608.0


def _tr_body(g_ref, t_ref):
    for yq in range(YB // 4):
        parts = [
            jnp.transpose(g_ref[0, yq * 4 + p], (1, 0)) for p in range(4)
        ]
        t_ref[yq] = jnp.concatenate(parts, axis=1)


def _idx_body(p_ref, o_ref):
    def rnd(t):
        t = jnp.clip(t * (GS - 1.0), 0.0, GS - 1.0)
        return (t + _RND) - _RND

    for r in range(IRB):
        sl = pl.ds(r * 128, 128)
        x = rnd(p_ref[0:1, sl])
        y = rnd(p_ref[1:2, sl])
        z = rnd(p_ref[2:3, sl])
        yq = jnp.floor(y * 0.25)
        yr = y - yq * 4.0
        o_ref[r : r + 1, :] = (
            x * 16384.0 + yq * 512.0 + z * 4.0 + yr
        ).astype(jnp.int32)


def _sc_body(idx_hbm, table_hbm, out_hbm, idx_v, rows_v,
             si0, si1, sg0, sg1, so0, so1):
    wid = lax.axis_index("s") * NC + lax.axis_index("c")
    sem_in = (si0, si1)
    sem_g = (sg0, sg1)
    sem_out = (so0, so1)

    def in_copy(k, s):
        return pltpu.make_async_copy(
            idx_hbm.at[pl.ds((wid + k * NW) * NIDX, NIDX)],
            idx_v.at[s],
            sem_in[s],
        )

    def gather_copies(s):
        return [
            pltpu.make_async_copy(
                table_hbm.at[idx_v.at[s, j]],
                rows_v.at[s, pl.ds(j * 128, 128)],
                sem_g[s],
            )
            for j in range(NIDX)
        ]

    def out_copy(k, s):
        return pltpu.make_async_copy(
            rows_v.at[s], out_hbm.at[pl.ds((wid + k * NW) * C, C)], sem_out[s]
        )

    def drain_out(s):
        pltpu.make_async_copy(
            out_hbm.at[pl.ds(0, C)], rows_v.at[s], sem_out[s]
        ).wait()

    def valid(k):
        return wid + k * NW < NCHUNKS

    in_copy(0, 0).start()
    in_copy(1, 1).start()

    def outer(io, carry):
        for b in range(2):
            k = io * 2 + b

            @pl.when(valid(k))
            def _():
                in_copy(k, b).wait()

                @pl.when(io > 0)
                def _():
                    drain_out(b)

                for cp in gather_copies(b):
                    cp.start()
                for cp in gather_copies(b):
                    cp.wait()

                @pl.when(valid(k + 2))
                def _():
                    in_copy(k + 2, b).start()

                out_copy(k, b).start()

        return carry

    lax.fori_loop(0, NMAXH, outer, 0)

    for b in range(2):
        drain_out(b)


def _run(points, grid):
    g2 = jnp.transpose(grid, (0, 1, 3, 2))
    pts_t = jnp.transpose(points, (1, 0))

    table = pl.pallas_call(
        _tr_body,
        grid=(GS, GS // YB),
        in_specs=[
            pl.BlockSpec((1, YB, F, GS), lambda i, j: (i, j, 0, 0)),
        ],
        out_specs=pl.BlockSpec(
            (YB // 4, GS, 4 * F), lambda i, j: (i * (GS // YB) + j, 0, 0)
        ),
        out_shape=jax.ShapeDtypeStruct((GS * GS // 4, GS, 4 * F), jnp.float32),
    )(g2)
    table = table.reshape(V, F)

    idx = pl.pallas_call(
        _idx_body,
        grid=(NPROWS_PAD // IRB,),
        in_specs=[pl.BlockSpec((3, IRB * 128), lambda i: (0, i))],
        out_specs=pl.BlockSpec((IRB, 128), lambda i: (i, 0)),
        out_shape=jax.ShapeDtypeStruct((NPROWS_PAD, 128), jnp.int32),
    )(pts_t)

    mesh = plsc.VectorSubcoreMesh(core_axis_name="c", subcore_axis_name="s")
    run = pl.kernel(
        _sc_body,
        out_type=jax.ShapeDtypeStruct((N, F), jnp.float32),
        mesh=mesh,
        compiler_params=pltpu.CompilerParams(
            needs_layout_passes=False, use_tc_tiling_on_sc=False
        ),
        scratch_types=[
            pltpu.VMEM((2, NIDX, 128), jnp.int32),
            pltpu.VMEM((2, C, F), jnp.float32),
            pltpu.SemaphoreType.DMA,
            pltpu.SemaphoreType.DMA,
            pltpu.SemaphoreType.DMA,
            pltpu.SemaphoreType.DMA,
            pltpu.SemaphoreType.DMA,
            pltpu.SemaphoreType.DMA,
        ],
    )
    return run(idx, table)


_run_jit = jax.jit(_run)


def kernel(points, grid):
    return _run_jit(points, grid)


# speedup vs baseline: 2.1951x; 1.1607x over previous
import jax
import jax.numpy as jnp
from jax import lax
from jax.experimental import pallas as pl
from jax.experimental.pallas import tpu as pltpu
from jax.experimental.pallas import tpu_sc as plsc

GS = 128
F = 32
N = 2000000
V = GS * GS * GS

NC = 2
NS = 16
NW = NC * NS

C = 640
NIDX = C // 128
NCHUNKS = N // C
NMAXH = (NCHUNKS + 2 * NW - 1) // (2 * NW)

YB = 32
PB = 16000
NPB = N // PB
CPB = PB // C
_RND = 8388608.0


def _tr_body(g_ref, t_ref):
    for yq in range(YB // 4):
        parts = [
            jnp.transpose(g_ref[0, yq * 4 + p], (1, 0)) for p in range(4)
        ]
        t_ref[yq] = jnp.concatenate(parts, axis=1)


def _idx_body(p_ref, o_ref):
    def rnd(t):
        t = jnp.clip(t * (GS - 1.0), 0.0, GS - 1.0)
        return (t + _RND) - _RND

    x = rnd(p_ref[0:1, :])
    y = rnd(p_ref[1:2, :])
    z = rnd(p_ref[2:3, :])
    yq = jnp.floor(y * 0.25)
    yr = y - yq * 4.0
    o_ref[0] = (x * 16384.0 + yq * 512.0 + z * 4.0 + yr).astype(jnp.int32)


def _sc_body(idx_hbm, table_hbm, out_hbm, idx_v, rows_v,
             si0, si1, sg0, sg1, so0, so1):
    wid = lax.axis_index("s") * NC + lax.axis_index("c")
    sem_in = (si0, si1)
    sem_g = (sg0, sg1)
    sem_out = (so0, so1)

    def in_copy(k, s):
        c = wid + k * NW
        return pltpu.make_async_copy(
            idx_hbm.at[c // CPB, 0, pl.ds((c % CPB) * C, C)],
            idx_v.at[s],
            sem_in[s],
        )

    def gather_copies(s):
        return [
            pltpu.make_async_copy(
                table_hbm.at[idx_v.at[s, pl.ds(j * 128, 128)]],
                rows_v.at[s, pl.ds(j * 128, 128)],
                sem_g[s],
            )
            for j in range(NIDX)
        ]

    def out_copy(k, s):
        return pltpu.make_async_copy(
            rows_v.at[s], out_hbm.at[pl.ds((wid + k * NW) * C, C)], sem_out[s]
        )

    def drain_out(s):
        pltpu.make_async_copy(
            out_hbm.at[pl.ds(0, C)], rows_v.at[s], sem_out[s]
        ).wait()

    def valid(k):
        return wid + k * NW < NCHUNKS

    in_copy(0, 0).start()
    in_copy(1, 1).start()

    def outer(io, carry):
        for b in range(2):
            k = io * 2 + b

            @pl.when(valid(k))
            def _():
                in_copy(k, b).wait()

                @pl.when(io > 0)
                def _():
                    drain_out(b)

                for cp in gather_copies(b):
                    cp.start()
                for cp in gather_copies(b):
                    cp.wait()

                @pl.when(valid(k + 2))
                def _():
                    in_copy(k + 2, b).start()

                out_copy(k, b).start()

        return carry

    lax.fori_loop(0, NMAXH, outer, 0)

    for b in range(2):
        drain_out(b)


def _run(points, grid):
    g2 = jnp.transpose(grid, (0, 1, 3, 2))
    pts_t = jnp.transpose(points, (1, 0))

    table = pl.pallas_call(
        _tr_body,
        grid=(GS, GS // YB),
        in_specs=[
            pl.BlockSpec((1, YB, F, GS), lambda i, j: (i, j, 0, 0)),
        ],
        out_specs=pl.BlockSpec(
            (YB // 4, GS, 4 * F), lambda i, j: (i * (GS // YB) + j, 0, 0)
        ),
        out_shape=jax.ShapeDtypeStruct((GS * GS // 4, GS, 4 * F), jnp.float32),
    )(g2)
    table = table.reshape(V, F)

    idx = pl.pallas_call(
        _idx_body,
        grid=(NPB,),
        in_specs=[pl.BlockSpec((3, PB), lambda i: (0, i))],
        out_specs=pl.BlockSpec((1, 1, PB), lambda i: (i, 0, 0)),
        out_shape=jax.ShapeDtypeStruct((NPB, 1, PB), jnp.int32),
    )(pts_t)

    mesh = plsc.VectorSubcoreMesh(core_axis_name="c", subcore_axis_name="s")
    run = pl.kernel(
        _sc_body,
        out_type=jax.ShapeDtypeStruct((N, F), jnp.float32),
        mesh=mesh,
        compiler_params=pltpu.CompilerParams(
            needs_layout_passes=False, use_tc_tiling_on_sc=False
        ),
        scratch_types=[
            pltpu.VMEM((2, C), jnp.int32),
            pltpu.VMEM((2, C, F), jnp.float32),
            pltpu.SemaphoreType.DMA,
            pltpu.SemaphoreType.DMA,
            pltpu.SemaphoreType.DMA,
            pltpu.SemaphoreType.DMA,
            pltpu.SemaphoreType.DMA,
            pltpu.SemaphoreType.DMA,
        ],
    )
    return run(idx, table)


_run_jit = jax.jit(_run)


def kernel(points, grid):
    return _run_jit(points, grid)


# speedup vs baseline: 2.3042x; 1.0497x over previous
import jax
import jax.numpy as jnp
from jax import lax
from jax.experimental import pallas as pl
from jax.experimental.pallas import tpu as pltpu
from jax.experimental.pallas import tpu_sc as plsc

GS = 128
F = 32
N = 2000000
V = GS * GS * GS

NC = 2
NS = 16
NW = NC * NS

C = 640
NIDX = C // 128
NCHUNKS = N // C
NMAXH = (NCHUNKS + 2 * NW - 1) // (2 * NW)

YB = 64
PB = 16000
NPB = N // PB
CPB = PB // C
_RND = 8388608.0


def _tr_body(g_ref, t_ref):
    for yq in range(YB // 4):
        parts = [
            jnp.transpose(g_ref[0, yq * 4 + p], (1, 0)) for p in range(4)
        ]
        t_ref[yq] = jnp.concatenate(parts, axis=1)


def _idx_body(p_ref, o_ref):
    def rnd(t):
        t = jnp.clip(t * (GS - 1.0), 0.0, GS - 1.0)
        return (t + _RND) - _RND

    x = rnd(p_ref[0:1, :])
    y = rnd(p_ref[1:2, :])
    z = rnd(p_ref[2:3, :])
    yq = jnp.floor(y * 0.25)
    yr = y - yq * 4.0
    o_ref[0] = (x * 16384.0 + yq * 512.0 + z * 4.0 + yr).astype(jnp.int32)


def _sc_body(idx_hbm, table_hbm, out_hbm, idx_v, rows_v,
             si0, si1, sg0, sg1, so0, so1):
    wid = lax.axis_index("s") * NC + lax.axis_index("c")
    sem_in = (si0, si1)
    sem_g = (sg0, sg1)
    sem_out = (so0, so1)

    def in_copy(k, s):
        c = wid + k * NW
        return pltpu.make_async_copy(
            idx_hbm.at[c // CPB, 0, pl.ds((c % CPB) * C, C)],
            idx_v.at[s],
            sem_in[s],
        )

    def gather_copies(s):
        return [
            pltpu.make_async_copy(
                table_hbm.at[idx_v.at[s, pl.ds(j * 128, 128)]],
                rows_v.at[s, pl.ds(j * 128, 128)],
                sem_g[s],
            )
            for j in range(NIDX)
        ]

    def out_copy(k, s):
        return pltpu.make_async_copy(
            rows_v.at[s], out_hbm.at[pl.ds((wid + k * NW) * C, C)], sem_out[s]
        )

    def drain_out(s):
        pltpu.make_async_copy(
            out_hbm.at[pl.ds(0, C)], rows_v.at[s], sem_out[s]
        ).wait()

    def valid(k):
        return wid + k * NW < NCHUNKS

    in_copy(0, 0).start()
    in_copy(1, 1).start()

    def outer(io, carry):
        for b in range(2):
            k = io * 2 + b

            @pl.when(valid(k))
            def _():
                in_copy(k, b).wait()

                @pl.when(io > 0)
                def _():
                    drain_out(b)

                for cp in gather_copies(b):
                    cp.start()
                for cp in gather_copies(b):
                    cp.wait()

                @pl.when(valid(k + 2))
                def _():
                    in_copy(k + 2, b).start()

                out_copy(k, b).start()

        return carry

    lax.fori_loop(0, NMAXH, outer, 0)

    for b in range(2):
        drain_out(b)


def _run(points, grid):
    g2 = jnp.transpose(grid, (0, 1, 3, 2))
    pts_t = jnp.transpose(points, (1, 0))

    table = pl.pallas_call(
        _tr_body,
        grid=(GS, GS // YB),
        in_specs=[
            pl.BlockSpec((1, YB, F, GS), lambda i, j: (i, j, 0, 0)),
        ],
        out_specs=pl.BlockSpec(
            (YB // 4, GS, 4 * F), lambda i, j: (i * (GS // YB) + j, 0, 0)
        ),
        out_shape=jax.ShapeDtypeStruct((GS * GS // 4, GS, 4 * F), jnp.float32),
    )(g2)
    table = table.reshape(V, F)

    idx = pl.pallas_call(
        _idx_body,
        grid=(NPB,),
        in_specs=[pl.BlockSpec((3, PB), lambda i: (0, i))],
        out_specs=pl.BlockSpec((1, 1, PB), lambda i: (i, 0, 0)),
        out_shape=jax.ShapeDtypeStruct((NPB, 1, PB), jnp.int32),
    )(pts_t)

    mesh = plsc.VectorSubcoreMesh(core_axis_name="c", subcore_axis_name="s")
    run = pl.kernel(
        _sc_body,
        out_type=jax.ShapeDtypeStruct((N, F), jnp.float32),
        mesh=mesh,
        compiler_params=pltpu.CompilerParams(
            needs_layout_passes=False, use_tc_tiling_on_sc=False
        ),
        scratch_types=[
            pltpu.VMEM((2, C), jnp.int32),
            pltpu.VMEM((2, C, F), jnp.float32),
            pltpu.SemaphoreType.DMA,
            pltpu.SemaphoreType.DMA,
            pltpu.SemaphoreType.DMA,
            pltpu.SemaphoreType.DMA,
            pltpu.SemaphoreType.DMA,
            pltpu.SemaphoreType.DMA,
        ],
    )
    return run(idx, table)


_run_jit = jax.jit(_run)


def kernel(points, grid):
    return _run_jit(points, grid)


# speedup vs baseline: 2.3259x; 1.0094x over previous
import jax
import jax.numpy as jnp
from jax import lax
from jax.experimental import pallas as pl
from jax.experimental.pallas import tpu as pltpu
from jax.experimental.pallas import tpu_sc as plsc

GS = 128
F = 32
N = 2000000
V = GS * GS * GS

NC = 2
NS = 16
NW = NC * NS

C = 640
NIDX = C // 128
NCHUNKS = N // C
NMAXH = (NCHUNKS + 2 * NW - 1) // (2 * NW)

YB = 128
PB = 16000
NPB = N // PB
CPB = PB // C
_RND = 8388608.0


def _tr_body(g_ref, t_ref):
    for yq in range(YB // 4):
        parts = [
            jnp.transpose(g_ref[0, yq * 4 + p], (1, 0)) for p in range(4)
        ]
        t_ref[yq] = jnp.concatenate(parts, axis=1)


def _idx_body(p_ref, o_ref):
    def rnd(t):
        t = jnp.clip(t * (GS - 1.0), 0.0, GS - 1.0)
        return (t + _RND) - _RND

    x = rnd(p_ref[0:1, :])
    y = rnd(p_ref[1:2, :])
    z = rnd(p_ref[2:3, :])
    yq = jnp.floor(y * 0.25)
    yr = y - yq * 4.0
    o_ref[0] = (x * 16384.0 + yq * 512.0 + z * 4.0 + yr).astype(jnp.int32)


def _sc_body(idx_hbm, table_hbm, out_hbm, idx_v, rows_v,
             si0, si1, sg0, sg1, so0, so1):
    wid = lax.axis_index("s") * NC + lax.axis_index("c")
    sem_in = (si0, si1)
    sem_g = (sg0, sg1)
    sem_out = (so0, so1)

    def in_copy(k, s):
        c = wid + k * NW
        return pltpu.make_async_copy(
            idx_hbm.at[c // CPB, 0, pl.ds((c % CPB) * C, C)],
            idx_v.at[s],
            sem_in[s],
        )

    def gather_copies(s):
        return [
            pltpu.make_async_copy(
                table_hbm.at[idx_v.at[s, pl.ds(j * 128, 128)]],
                rows_v.at[s, pl.ds(j * 128, 128)],
                sem_g[s],
            )
            for j in range(NIDX)
        ]

    def out_copy(k, s):
        return pltpu.make_async_copy(
            rows_v.at[s], out_hbm.at[pl.ds((wid + k * NW) * C, C)], sem_out[s]
        )

    def drain_out(s):
        pltpu.make_async_copy(
            out_hbm.at[pl.ds(0, C)], rows_v.at[s], sem_out[s]
        ).wait()

    def valid(k):
        return wid + k * NW < NCHUNKS

    in_copy(0, 0).start()
    in_copy(1, 1).start()

    def outer(io, carry):
        for b in range(2):
            k = io * 2 + b

            @pl.when(valid(k))
            def _():
                in_copy(k, b).wait()

                @pl.when(io > 0)
                def _():
                    drain_out(b)

                for cp in gather_copies(b):
                    cp.start()
                for cp in gather_copies(b):
                    cp.wait()

                @pl.when(valid(k + 2))
                def _():
                    in_copy(k + 2, b).start()

                out_copy(k, b).start()

        return carry

    lax.fori_loop(0, NMAXH, outer, 0)

    for b in range(2):
        drain_out(b)


def _run(points, grid):
    g2 = jnp.transpose(grid, (0, 1, 3, 2))
    pts_t = jnp.transpose(points, (1, 0))

    table = pl.pallas_call(
        _tr_body,
        grid=(GS, GS // YB),
        in_specs=[
            pl.BlockSpec((1, YB, F, GS), lambda i, j: (i, j, 0, 0)),
        ],
        out_specs=pl.BlockSpec(
            (YB // 4, GS, 4 * F), lambda i, j: (i * (GS // YB) + j, 0, 0)
        ),
        out_shape=jax.ShapeDtypeStruct((GS * GS // 4, GS, 4 * F), jnp.float32),
    )(g2)
    table = table.reshape(V, F)

    idx = pl.pallas_call(
        _idx_body,
        grid=(NPB,),
        in_specs=[pl.BlockSpec((3, PB), lambda i: (0, i))],
        out_specs=pl.BlockSpec((1, 1, PB), lambda i: (i, 0, 0)),
        out_shape=jax.ShapeDtypeStruct((NPB, 1, PB), jnp.int32),
    )(pts_t)

    mesh = plsc.VectorSubcoreMesh(core_axis_name="c", subcore_axis_name="s")
    run = pl.kernel(
        _sc_body,
        out_type=jax.ShapeDtypeStruct((N, F), jnp.float32),
        mesh=mesh,
        compiler_params=pltpu.CompilerParams(
            needs_layout_passes=False, use_tc_tiling_on_sc=False
        ),
        scratch_types=[
            pltpu.VMEM((2, C), jnp.int32),
            pltpu.VMEM((2, C, F), jnp.float32),
            pltpu.SemaphoreType.DMA,
            pltpu.SemaphoreType.DMA,
            pltpu.SemaphoreType.DMA,
            pltpu.SemaphoreType.DMA,
            pltpu.SemaphoreType.DMA,
            pltpu.SemaphoreType.DMA,
        ],
    )
    return run(idx, table)


_run_jit = jax.jit(_run)


def kernel(points, grid):
    return _run_jit(points, grid)
